# baseline (speedup 1.0000x reference)
import functools

import jax
import jax.numpy as jnp
from jax.experimental import pallas as pl
from jax.experimental.pallas import tpu as pltpu

_LRELU = 0.2
_NEG = -1e9


def _proj_kernel(h_ref, w_ref, mel_ref, mer_ref, feat_ref, u_ref, v_ref):
    feat = jnp.dot(h_ref[0], w_ref[...], preferred_element_type=jnp.float32)
    feat_ref[0] = feat
    el = jnp.dot(feat, mel_ref[...], preferred_element_type=jnp.float32)
    er = jnp.dot(feat, mer_ref[...], preferred_element_type=jnp.float32)
    u_ref[0] = jnp.exp2(0.8 * el)
    v_ref[0] = jnp.concatenate(
        [jnp.exp2(0.8 * er), jnp.exp2(_LRELU * er)], axis=1)


def _att_kernel(nheads, F, post_relu, feat_ref, a_ref, ai_ref, ao_ref,
                g_ref, kt_ref, vp_ref, out_ref):
    g = g_ref[0]
    n_src = feat_ref.shape[1]
    ones = jnp.ones((n_src, 1), jnp.float32)
    vfeat = [[jnp.concatenate(
        [feat_ref[0, :, h * F:(h + 1) * F], ones], axis=1)
        * vp_ref[0, :, (t * nheads + h):(t * nheads + h) + 1]
        for h in range(nheads)] for t in range(3)]
    acc = None
    for t, aref in enumerate((a_ref, ai_ref, ao_ref)):
        a = aref[0]
        for h in range(nheads):
            c = t * nheads + h
            p = jnp.maximum(g[:, c:c + 1] * kt_ref[0, c:c + 1, :], 1.0) * a
            os = jnp.dot(p, vfeat[t][h], preferred_element_type=jnp.float32)
            o = os[:, :F] / os[:, F:F + 1]
            acc = o if acc is None else acc + o
    acc = acc * (1.0 / (3 * nheads))
    if post_relu:
        acc = jnp.maximum(acc, 0.0)
    out_ref[0] = acc


def _score_mat(a):
    _, nh, F = a.shape
    cols = []
    for t in range(3):
        for hh in range(nh):
            col = jnp.zeros((nh, F), jnp.float32).at[hh].set(a[t, hh])
            cols.append(col.reshape(nh * F))
    cols.append(jnp.zeros((nh * F,), jnp.float32))
    return jnp.stack(cols, axis=1)


def _gat_layer(h, adj, adj_in, adj_out, W, al, ar, post_relu, ti):
    B, N, din = h.shape
    _, nh, F = al.shape
    HF = nh * F
    log2e = 1.4426950408889634
    mel, mer = _score_mat(al) * log2e, _score_mat(ar) * log2e
    feat, elv, erv = pl.pallas_call(
        _proj_kernel,
        grid=(B,),
        in_specs=[
            pl.BlockSpec((1, N, din), lambda b: (b, 0, 0)),
            pl.BlockSpec((din, HF), lambda b: (0, 0)),
            pl.BlockSpec((HF, 16), lambda b: (0, 0)),
            pl.BlockSpec((HF, 16), lambda b: (0, 0)),
        ],
        out_specs=[
            pl.BlockSpec((1, N, HF), lambda b: (b, 0, 0)),
            pl.BlockSpec((1, N, 16), lambda b: (b, 0, 0)),
            pl.BlockSpec((1, N, 32), lambda b: (b, 0, 0)),
        ],
        out_shape=[
            jax.ShapeDtypeStruct((B, N, HF), jnp.float32),
            jax.ShapeDtypeStruct((B, N, 16), jnp.float32),
            jax.ShapeDtypeStruct((B, N, 32), jnp.float32),
        ],
    )(h, W, mel, mer)
    kt = jnp.swapaxes(erv[:, :, :16], 1, 2)
    vp = erv[:, :, 16:]
    out = pl.pallas_call(
        functools.partial(_att_kernel, nh, F, post_relu),
        grid=(B, N // ti),
        in_specs=[
            pl.BlockSpec((1, N, HF), lambda b, i: (b, 0, 0)),
            pl.BlockSpec((1, ti, N), lambda b, i: (b, i, 0)),
            pl.BlockSpec((1, ti, N), lambda b, i: (b, i, 0)),
            pl.BlockSpec((1, ti, N), lambda b, i: (b, i, 0)),
            pl.BlockSpec((1, ti, 16), lambda b, i: (b, i, 0)),
            pl.BlockSpec((1, 16, N), lambda b, i: (b, 0, 0)),
            pl.BlockSpec((1, N, 16), lambda b, i: (b, 0, 0)),
        ],
        out_specs=pl.BlockSpec((1, ti, F), lambda b, i: (b, i, 0)),
        out_shape=jax.ShapeDtypeStruct((B, N, F), jnp.float32),
        compiler_params=pltpu.CompilerParams(
            dimension_semantics=("parallel", "parallel")),
    )(feat, adj, adj_in, adj_out, elv, kt, vp)
    return out


def kernel(inputs, adj, adj_in, adj_out, W1, al1, ar1, W2, al2, ar2):
    h1 = _gat_layer(inputs, adj, adj_in, adj_out, W1, al1, ar1, True, 512)
    return _gat_layer(h1, adj, adj_in, adj_out, W2, al2, ar2, False, 512)


# speedup vs baseline: 1.2606x; 1.2606x over previous
import functools

import jax
import jax.numpy as jnp
from jax.experimental import pallas as pl
from jax.experimental.pallas import tpu as pltpu

_LRELU = 0.2
_NEG = -1e9


def _proj_kernel(h_ref, w_ref, mel_ref, mer_ref, feat_ref, u_ref, v_ref):
    feat = jnp.dot(h_ref[0], w_ref[...], preferred_element_type=jnp.float32)
    feat_ref[0] = feat
    el = jnp.dot(feat, mel_ref[...], preferred_element_type=jnp.float32)
    er = jnp.dot(feat, mer_ref[...], preferred_element_type=jnp.float32)
    u_ref[0] = jnp.concatenate([jnp.exp2(el), jnp.exp2(_LRELU * el)], axis=1)
    v_ref[0] = jnp.concatenate([jnp.exp2(er), jnp.exp2(_LRELU * er)], axis=1)


def _att_kernel(nheads, F, post_relu, feat_ref, a_ref, ai_ref, ao_ref,
                u_ref, vt_ref, out_ref):
    u = u_ref[0]
    n_src = feat_ref.shape[1]
    ones = jnp.ones((n_src, 1), jnp.float32)
    feataug = [jnp.concatenate(
        [feat_ref[0, :, h * F:(h + 1) * F], ones], axis=1)
        for h in range(nheads)]
    acc = None
    for t, aref in enumerate((a_ref, ai_ref, ao_ref)):
        a = aref[0]
        for h in range(nheads):
            c = t * nheads + h
            p = jnp.maximum(u[:, c:c + 1] * vt_ref[0, c:c + 1, :],
                            u[:, 16 + c:17 + c] * vt_ref[0, 16 + c:17 + c, :]
                            ) * a
            os = jnp.dot(p, feataug[h], preferred_element_type=jnp.float32)
            o = os[:, :F] / os[:, F:F + 1]
            acc = o if acc is None else acc + o
    acc = acc * (1.0 / (3 * nheads))
    if post_relu:
        acc = jnp.maximum(acc, 0.0)
    out_ref[0] = acc


def _score_mat(a):
    _, nh, F = a.shape
    cols = []
    for t in range(3):
        for hh in range(nh):
            col = jnp.zeros((nh, F), jnp.float32).at[hh].set(a[t, hh])
            cols.append(col.reshape(nh * F))
    cols.append(jnp.zeros((nh * F,), jnp.float32))
    return jnp.stack(cols, axis=1)


def _gat_layer(h, adj, adj_in, adj_out, W, al, ar, post_relu, ti):
    B, N, din = h.shape
    _, nh, F = al.shape
    HF = nh * F
    log2e = 1.4426950408889634
    mel, mer = _score_mat(al) * log2e, _score_mat(ar) * log2e
    feat, elv, erv = pl.pallas_call(
        _proj_kernel,
        grid=(B,),
        in_specs=[
            pl.BlockSpec((1, N, din), lambda b: (b, 0, 0)),
            pl.BlockSpec((din, HF), lambda b: (0, 0)),
            pl.BlockSpec((HF, 16), lambda b: (0, 0)),
            pl.BlockSpec((HF, 16), lambda b: (0, 0)),
        ],
        out_specs=[
            pl.BlockSpec((1, N, HF), lambda b: (b, 0, 0)),
            pl.BlockSpec((1, N, 32), lambda b: (b, 0, 0)),
            pl.BlockSpec((1, N, 32), lambda b: (b, 0, 0)),
        ],
        out_shape=[
            jax.ShapeDtypeStruct((B, N, HF), jnp.float32),
            jax.ShapeDtypeStruct((B, N, 32), jnp.float32),
            jax.ShapeDtypeStruct((B, N, 32), jnp.float32),
        ],
    )(h, W, mel, mer)
    ert = jnp.swapaxes(erv, 1, 2)
    out = pl.pallas_call(
        functools.partial(_att_kernel, nh, F, post_relu),
        grid=(B, N // ti),
        in_specs=[
            pl.BlockSpec((1, N, HF), lambda b, i: (b, 0, 0)),
            pl.BlockSpec((1, ti, N), lambda b, i: (b, i, 0)),
            pl.BlockSpec((1, ti, N), lambda b, i: (b, i, 0)),
            pl.BlockSpec((1, ti, N), lambda b, i: (b, i, 0)),
            pl.BlockSpec((1, ti, 32), lambda b, i: (b, i, 0)),
            pl.BlockSpec((1, 32, N), lambda b, i: (b, 0, 0)),
        ],
        out_specs=pl.BlockSpec((1, ti, F), lambda b, i: (b, i, 0)),
        out_shape=jax.ShapeDtypeStruct((B, N, F), jnp.float32),
        compiler_params=pltpu.CompilerParams(
            dimension_semantics=("parallel", "parallel")),
    )(feat, adj, adj_in, adj_out, elv, ert)
    return out


def kernel(inputs, adj, adj_in, adj_out, W1, al1, ar1, W2, al2, ar2):
    h1 = _gat_layer(inputs, adj, adj_in, adj_out, W1, al1, ar1, True, 512)
    return _gat_layer(h1, adj, adj_in, adj_out, W2, al2, ar2, False, 512)


# speedup vs baseline: 1.5272x; 1.2115x over previous
import functools

import jax
import jax.numpy as jnp
from jax.experimental import pallas as pl
from jax.experimental.pallas import tpu as pltpu

_LRELU = 0.2


def _exp_tables(feat, mel_ref, mer_ref):
    el = jnp.dot(feat, mel_ref[...], preferred_element_type=jnp.float32)
    er = jnp.dot(feat, mer_ref[...], preferred_element_type=jnp.float32)
    u = jnp.concatenate([jnp.exp2(el), jnp.exp2(_LRELU * el)], axis=1)
    v = jnp.concatenate([jnp.exp2(er), jnp.exp2(_LRELU * er)], axis=1)
    return u, jnp.swapaxes(v, 0, 1)


def _att_layer(nh, F, ti, arefs, feat, u, vt):
    n = feat.shape[0]
    ones = jnp.ones((n, 1), jnp.float32)
    feataug = [jnp.concatenate([feat[:, h * F:(h + 1) * F], ones], axis=1)
               for h in range(nh)]
    rows = []
    for i0 in range(0, n, ti):
        ui = u[i0:i0 + ti, :]
        acc = None
        for t, aref in enumerate(arefs):
            a = aref[0, i0:i0 + ti, :]
            for h in range(nh):
                c = t * nh + h
                p = jnp.maximum(ui[:, c:c + 1] * vt[c:c + 1, :],
                                ui[:, 16 + c:17 + c] * vt[16 + c:17 + c, :]
                                ) * a
                os = jnp.dot(p, feataug[h],
                             preferred_element_type=jnp.float32)
                o = os[:, :F] / os[:, F:F + 1]
                acc = o if acc is None else acc + o
        rows.append(acc * (1.0 / (3 * nh)))
    return jnp.concatenate(rows, axis=0) if len(rows) > 1 else rows[0]


def _mega_kernel(nh, F1, F2, ti,
                 x_ref, a_ref, ai_ref, ao_ref,
                 w1_ref, mel1_ref, mer1_ref,
                 w2_ref, mel2_ref, mer2_ref, out_ref):
    arefs = (a_ref, ai_ref, ao_ref)
    feat1 = jnp.dot(x_ref[0], w1_ref[...], preferred_element_type=jnp.float32)
    u1, vt1 = _exp_tables(feat1, mel1_ref, mer1_ref)
    h1 = jnp.maximum(_att_layer(nh, F1, ti, arefs, feat1, u1, vt1), 0.0)
    feat2 = jnp.dot(h1, w2_ref[...], preferred_element_type=jnp.float32)
    u2, vt2 = _exp_tables(feat2, mel2_ref, mer2_ref)
    out_ref[0] = _att_layer(nh, F2, ti, arefs, feat2, u2, vt2)


def _score_mat(a):
    _, nh, F = a.shape
    cols = []
    for t in range(3):
        for hh in range(nh):
            col = jnp.zeros((nh, F), jnp.float32).at[hh].set(a[t, hh])
            cols.append(col.reshape(nh * F))
    cols.append(jnp.zeros((nh * F,), jnp.float32))
    return jnp.stack(cols, axis=1) * 1.4426950408889634


def kernel(inputs, adj, adj_in, adj_out, W1, al1, ar1, W2, al2, ar2):
    B, N, din = inputs.shape
    _, nh, F1 = al1.shape
    F2 = al2.shape[2]
    mel1, mer1 = _score_mat(al1), _score_mat(ar1)
    mel2, mer2 = _score_mat(al2), _score_mat(ar2)
    full = lambda shape: pl.BlockSpec(shape, lambda b: (0,) * len(shape))
    return pl.pallas_call(
        functools.partial(_mega_kernel, nh, F1, F2, 512),
        grid=(B,),
        in_specs=[
            pl.BlockSpec((1, N, din), lambda b: (b, 0, 0)),
            pl.BlockSpec((1, N, N), lambda b: (b, 0, 0)),
            pl.BlockSpec((1, N, N), lambda b: (b, 0, 0)),
            pl.BlockSpec((1, N, N), lambda b: (b, 0, 0)),
            full((din, nh * F1)),
            full((nh * F1, 16)),
            full((nh * F1, 16)),
            full((F1, nh * F2)),
            full((nh * F2, 16)),
            full((nh * F2, 16)),
        ],
        out_specs=pl.BlockSpec((1, N, F2), lambda b: (b, 0, 0)),
        out_shape=jax.ShapeDtypeStruct((B, N, F2), jnp.float32),
        compiler_params=pltpu.CompilerParams(
            dimension_semantics=("parallel",)),
    )(inputs, adj, adj_in, adj_out, W1, mel1, mer1, W2, mel2, mer2)
